# baseline (speedup 1.0000x reference)
import functools

import jax
import jax.numpy as jnp
from jax import lax
from jax.experimental import pallas as pl
from jax.experimental.pallas import tpu as pltpu
from jax.experimental.pallas import tpu_sc as plsc

_FEATURE_LAYERS = 5
_B = 128
_C = 8192
_E = 3000
_NL = 2 * _C
_K = 10
_G = 2

_NC, _NS, _L = 2, 16, 16
_NW = _NC * _NS
_ROWS_PER_W = _B // _NW

_CHUNKS_PER_GRP = 16
_GRPS = _C // (_L * _CHUNKS_PER_GRP)
_FULL_CHUNKS = (_E - 8) // _L


def _mm_bias_body(n_real, block_n, x_ref, w_ref, b_ref, o_ref):
    acc = (
        jnp.dot(x_ref[...], w_ref[...], preferred_element_type=jnp.float32)
        + b_ref[...]
    )
    if n_real is not None:
        col = (pl.program_id(0) * block_n
               + lax.broadcasted_iota(jnp.int32, acc.shape, 1))
        acc = jnp.where(col < n_real, acc, 0.0)
    o_ref[...] = acc


def _matmul_bias(x, w, b, block_n, n_pad=None):
    m, k = x.shape
    n = w.shape[1]
    n_out = n if n_pad is None else n_pad
    n_real = None if n_pad is None else n
    return pl.pallas_call(
        functools.partial(_mm_bias_body, n_real, block_n),
        grid=(pl.cdiv(n_out, block_n),),
        in_specs=[
            pl.BlockSpec((m, k), lambda j: (0, 0)),
            pl.BlockSpec((k, block_n), lambda j: (0, j)),
            pl.BlockSpec((1, block_n), lambda j: (0, j)),
        ],
        out_specs=pl.BlockSpec((m, block_n), lambda j: (0, j)),
        out_shape=jax.ShapeDtypeStruct((m, n_out), jnp.float32),
    )(x, w, b.reshape(1, n))


def _sc_body(logits_hbm, gy_hbm, emb_hbm, table_hbm, out_hbm,
             row_v, gy_v, cand_v, rows_v, emb_v, out_v, sem):
    wid = lax.axis_index("s") * _NC + lax.axis_index("c")
    base = wid * _ROWS_PER_W
    lane = lax.iota(jnp.int32, _L)
    lane0 = lane == 0
    k_mask = lane < _K
    neg_inf = jnp.full((_L,), -jnp.inf, jnp.float32)

    pltpu.sync_copy(gy_hbm, gy_v)

    for r in range(_ROWS_PER_W):
        b = base + r
        pltpu.sync_copy(logits_hbm.at[pl.ds(b, 1)], row_v)
        pltpu.sync_copy(emb_hbm.at[pl.ds(b, 1)], emb_v)

        def grp_body(g, carry):
            cval, cidx, thr = carry
            gbase = g * (_L * _CHUNKS_PER_GRP)
            m = row_v[0, pl.ds(gbase, _L)]
            for j in range(1, _CHUNKS_PER_GRP):
                m = jnp.maximum(m, row_v[0, pl.ds(gbase + j * _L, _L)])
            gmax = jnp.max(m)

            def merge(c3):
                cv, ci, _ = c3
                for j in range(_CHUNKS_PER_GRP):
                    v = row_v[0, pl.ds(gbase + j * _L, _L)]
                    vi = gbase + j * _L + lane
                    sv, si = plsc.sort_key_val(v, vi, descending=True)
                    rv = lax.rev(sv, (0,))
                    ri = lax.rev(si, (0,))
                    take = rv > cv
                    nv = jnp.where(take, rv, cv)
                    ni = jnp.where(take, ri, ci)
                    cv, ci = plsc.sort_key_val(nv, ni, descending=True)
                new_thr = jnp.min(jnp.where(k_mask, cv, jnp.inf))
                return cv, ci, new_thr

            return lax.cond(gmax > thr, merge, lambda c3: c3,
                            (cval, cidx, thr))

        cval, cidx, thr = lax.fori_loop(
            0, _GRPS, grp_body,
            (neg_inf, jnp.zeros((_L,), jnp.int32), -jnp.inf))

        safe_idx = jnp.where(k_mask, cidx, 0)
        ev = plsc.load_gather(gy_v, [safe_idx * 2])
        ov = plsc.load_gather(gy_v, [safe_idx * 2 + 1])
        plsc.store_scatter(cand_v, [lane * 2], ev, mask=k_mask)
        plsc.store_scatter(cand_v, [lane * 2 + 1], ov, mask=k_mask)

        pltpu.async_copy(table_hbm.at[cand_v], rows_v, sem).wait()

        def dot_body(kk, accs):
            e = emb_v[0, pl.ds(kk * _L, _L)]
            return tuple(
                accs[c] + rows_v[c, pl.ds(kk * _L, _L)] * e
                for c in range(_G * _K)
            )

        accs = lax.fori_loop(
            0, _FULL_CHUNKS, dot_body,
            tuple(jnp.zeros((_L,), jnp.float32) for _ in range(_G * _K)))

        e_tail = emb_v[0, pl.ds(_E - 8, _L)]
        tail_idx = jnp.minimum((_E - 8) + lane, _E - 1)
        for c in range(_G * _K):
            a = plsc.load_gather(
                rows_v, [jnp.full((_L,), c, jnp.int32), tail_idx])
            s = jnp.sum(accs[c] + a * e_tail)
            plsc.store_scatter(
                out_v,
                [jnp.full((_L,), r * _G * _K + c, jnp.int32)],
                jnp.full((_L,), s),
                mask=lane0)

    pltpu.sync_copy(
        out_v, out_hbm.at[pl.ds(base * _G * _K, _ROWS_PER_W * _G * _K)])


_sc_route_score = functools.partial(
    pl.kernel,
    mesh=plsc.VectorSubcoreMesh(core_axis_name="c", subcore_axis_name="s"),
    out_type=jax.ShapeDtypeStruct((_B * _G * _K,), jnp.float32),
    scratch_types=[
        pltpu.VMEM((1, _C), jnp.float32),
        pltpu.VMEM((_NL,), jnp.int32),
        pltpu.VMEM((_G * _K,), jnp.int32),
        pltpu.VMEM((_G * _K, _E), jnp.float32),
        pltpu.VMEM((1, _E + 72), jnp.float32),
        pltpu.VMEM((_ROWS_PER_W * _G * _K,), jnp.float32),
        pltpu.SemaphoreType.DMA,
    ],
    compiler_params=pltpu.CompilerParams(
        needs_layout_passes=False, use_tc_tiling_on_sc=False),
)(_sc_body)


@jax.jit
def kernel(hidden_states, labels, W1, b1, W2, b2, group_y, embed_table):
    del labels
    cls_feats = jnp.concatenate(
        [hidden_states[-i][:, 0] for i in range(1, _FEATURE_LAYERS + 1)],
        axis=-1)
    logits = _matmul_bias(cls_feats, W1, b1, 1024)
    emb = _matmul_bias(logits, W2, b2, 512, n_pad=_E + 72)
    gy_flat = group_y.reshape(-1)
    out = _sc_route_score(logits, gy_flat, emb, embed_table)
    return out.reshape(_B, _G * _K)


# speedup vs baseline: 1.3784x; 1.3784x over previous
import functools

import jax
import jax.numpy as jnp
from jax import lax
from jax.experimental import pallas as pl
from jax.experimental.pallas import tpu as pltpu
from jax.experimental.pallas import tpu_sc as plsc

_FEATURE_LAYERS = 5
_B = 128
_C = 8192
_E = 3000
_NL = 2 * _C
_K = 10
_G = 2
_NCAND = _G * _K

_NC, _NS, _L = 2, 16, 16
_NW = _NC * _NS
_ROWS_PER_W = _B // _NW

_CHUNKS_PER_GRP = 16
_GRPS = _C // (_L * _CHUNKS_PER_GRP)


def _mm_bias_body(x_ref, w_ref, b_ref, o_ref):
    o_ref[...] = (
        jnp.dot(x_ref[...], w_ref[...], preferred_element_type=jnp.float32)
        + b_ref[...]
    )


def _matmul_bias(x, w, b, block_n):
    m, k = x.shape
    n = w.shape[1]
    return pl.pallas_call(
        _mm_bias_body,
        grid=(pl.cdiv(n, block_n),),
        in_specs=[
            pl.BlockSpec((m, k), lambda j: (0, 0)),
            pl.BlockSpec((k, block_n), lambda j: (0, j)),
            pl.BlockSpec((1, block_n), lambda j: (0, j)),
        ],
        out_specs=pl.BlockSpec((m, block_n), lambda j: (0, j)),
        out_shape=jax.ShapeDtypeStruct((m, n), jnp.float32),
    )(x, w, b.reshape(1, n))


_RB = 8


def _gather_dot_body(cand_ref, table_ref, emb_ref, o_ref, rows_v, sem):
    g = pl.program_id(0)

    def _copy(j):
        idx = cand_ref[(g * _RB) * _NCAND + j]
        return pltpu.make_async_copy(
            table_ref.at[pl.ds(idx, 1)], rows_v.at[pl.ds(j, 1)], sem)

    for j in range(_RB * _NCAND):
        _copy(j).start()
    for j in range(_RB * _NCAND):
        _copy(j).wait()
    for i in range(_RB):
        o_ref[pl.ds(i, 1), :] = lax.dot_general(
            emb_ref[pl.ds(i, 1), :], rows_v[pl.ds(i * _NCAND, _NCAND), :],
            (((1,), (1,)), ((), ())),
            preferred_element_type=jnp.float32)


def _gather_dot(cand, embed_table, emb):
    return pl.pallas_call(
        _gather_dot_body,
        grid_spec=pltpu.PrefetchScalarGridSpec(
            num_scalar_prefetch=1,
            grid=(_B // _RB,),
            in_specs=[
                pl.BlockSpec(memory_space=pl.ANY),
                pl.BlockSpec((_RB, _E), lambda g, c: (g, 0)),
            ],
            out_specs=pl.BlockSpec((_RB, _NCAND), lambda g, c: (g, 0)),
            scratch_shapes=[
                pltpu.VMEM((_RB * _NCAND, _E), jnp.float32),
                pltpu.SemaphoreType.DMA,
            ],
        ),
        out_shape=jax.ShapeDtypeStruct((_B, _NCAND), jnp.float32),
    )(cand, embed_table, emb)


def _sc_body(logits_hbm, gy_hbm, cand_hbm, row_v, gy_v, cand_v):
    wid = lax.axis_index("s") * _NC + lax.axis_index("c")
    base = wid * _ROWS_PER_W
    lane = lax.iota(jnp.int32, _L)
    k_mask = lane < _K
    neg_inf = jnp.full((_L,), -jnp.inf, jnp.float32)

    pltpu.sync_copy(gy_hbm, gy_v)

    for r in range(_ROWS_PER_W):
        b = base + r
        pltpu.sync_copy(logits_hbm.at[pl.ds(b, 1)], row_v)

        def grp_body(g, carry):
            cval, cidx, thr = carry
            gbase = g * (_L * _CHUNKS_PER_GRP)
            m = row_v[0, pl.ds(gbase, _L)]
            for j in range(1, _CHUNKS_PER_GRP):
                m = jnp.maximum(m, row_v[0, pl.ds(gbase + j * _L, _L)])
            gmax = jnp.max(m)

            def merge(c3):
                cv, ci, _ = c3
                for j in range(_CHUNKS_PER_GRP):
                    v = row_v[0, pl.ds(gbase + j * _L, _L)]
                    vi = gbase + j * _L + lane
                    sv, si = plsc.sort_key_val(v, vi, descending=True)
                    rv = lax.rev(sv, (0,))
                    ri = lax.rev(si, (0,))
                    take = rv > cv
                    nv = jnp.where(take, rv, cv)
                    ni = jnp.where(take, ri, ci)
                    cv, ci = plsc.sort_key_val(nv, ni, descending=True)
                new_thr = jnp.min(jnp.where(k_mask, cv, jnp.inf))
                return cv, ci, new_thr

            return lax.cond(gmax > thr, merge, lambda c3: c3,
                            (cval, cidx, thr))

        _, cidx, _ = lax.fori_loop(
            0, _GRPS, grp_body,
            (neg_inf, jnp.zeros((_L,), jnp.int32), -jnp.inf))

        safe_idx = jnp.where(k_mask, cidx, 0)
        ev = plsc.load_gather(gy_v, [safe_idx * 2])
        ov = plsc.load_gather(gy_v, [safe_idx * 2 + 1])
        plsc.store_scatter(cand_v, [r * _NCAND + lane * 2], ev, mask=k_mask)
        plsc.store_scatter(cand_v, [r * _NCAND + lane * 2 + 1], ov,
                           mask=k_mask)

    pltpu.sync_copy(
        cand_v, cand_hbm.at[pl.ds(base * _NCAND, _ROWS_PER_W * _NCAND)])


_sc_topk_route = functools.partial(
    pl.kernel,
    mesh=plsc.VectorSubcoreMesh(core_axis_name="c", subcore_axis_name="s"),
    out_type=jax.ShapeDtypeStruct((_B * _NCAND,), jnp.int32),
    scratch_types=[
        pltpu.VMEM((1, _C), jnp.float32),
        pltpu.VMEM((_NL,), jnp.int32),
        pltpu.VMEM((_ROWS_PER_W * _NCAND,), jnp.int32),
    ],
    compiler_params=pltpu.CompilerParams(
        needs_layout_passes=False, use_tc_tiling_on_sc=False),
)(_sc_body)


@jax.jit
def kernel(hidden_states, labels, W1, b1, W2, b2, group_y, embed_table):
    del labels
    cls_feats = jnp.concatenate(
        [hidden_states[-i][:, 0] for i in range(1, _FEATURE_LAYERS + 1)],
        axis=-1)
    logits = _matmul_bias(cls_feats, W1, b1, 1024)
    cand = _sc_topk_route(logits, group_y.reshape(-1))
    emb = _matmul_bias(logits, W2, b2, 512)
    return _gather_dot(cand, embed_table, emb)


# speedup vs baseline: 1.7199x; 1.2478x over previous
import functools

import jax
import jax.numpy as jnp
from jax import lax
from jax.experimental import pallas as pl
from jax.experimental.pallas import tpu as pltpu
from jax.experimental.pallas import tpu_sc as plsc

_FEATURE_LAYERS = 5
_B = 128
_C = 8192
_E = 3000
_NL = 2 * _C
_K = 10
_G = 2
_NCAND = _G * _K

_NC, _NS, _L = 2, 16, 16
_NW = _NC * _NS
_ROWS_PER_W = _B // _NW

_CHUNKS_PER_GRP = 16
_GRPS = _C // (_L * _CHUNKS_PER_GRP)


def _mm_bias_body(x_ref, w_ref, b_ref, o_ref):
    o_ref[...] = (
        jnp.dot(x_ref[...], w_ref[...], preferred_element_type=jnp.float32)
        + b_ref[...]
    )


def _matmul_bias(x, w, b, block_n):
    m, k = x.shape
    n = w.shape[1]
    return pl.pallas_call(
        _mm_bias_body,
        grid=(pl.cdiv(n, block_n),),
        in_specs=[
            pl.BlockSpec((m, k), lambda j: (0, 0)),
            pl.BlockSpec((k, block_n), lambda j: (0, j)),
            pl.BlockSpec((1, block_n), lambda j: (0, j)),
        ],
        out_specs=pl.BlockSpec((m, block_n), lambda j: (0, j)),
        out_shape=jax.ShapeDtypeStruct((m, n), jnp.float32),
    )(x, w, b.reshape(1, n))


def _mmT_bias_body(w_ref, x_ref, b_ref, o_ref):
    o_ref[...] = (
        lax.dot_general(w_ref[...], x_ref[...], (((1,), (1,)), ((), ())),
                        preferred_element_type=jnp.float32)
        + b_ref[...]
    )


def _matmulT_bias(wT, x, b, block_m):
    n, k = wT.shape
    m = x.shape[0]
    return pl.pallas_call(
        _mmT_bias_body,
        grid=(pl.cdiv(n, block_m),),
        in_specs=[
            pl.BlockSpec((block_m, k), lambda j: (j, 0)),
            pl.BlockSpec((m, k), lambda j: (0, 0)),
            pl.BlockSpec((block_m, 1), lambda j: (j, 0)),
        ],
        out_specs=pl.BlockSpec((block_m, m), lambda j: (j, 0)),
        out_shape=jax.ShapeDtypeStruct((n, m), jnp.float32),
    )(wT, x, b.reshape(n, 1))


_RB = 8


def _gather_dot_body(cand_ref, table_ref, emb_ref, o_ref, rows_v, sem):
    g = pl.program_id(0)

    def _copy(j):
        idx = cand_ref[(g * _RB) * _NCAND + j]
        return pltpu.make_async_copy(
            table_ref.at[pl.ds(idx, 1)], rows_v.at[pl.ds(j, 1)], sem)

    for j in range(_RB * _NCAND):
        _copy(j).start()
    for j in range(_RB * _NCAND):
        _copy(j).wait()
    for i in range(_RB):
        o_ref[pl.ds(i, 1), :] = lax.dot_general(
            emb_ref[pl.ds(i, 1), :], rows_v[pl.ds(i * _NCAND, _NCAND), :],
            (((1,), (1,)), ((), ())),
            preferred_element_type=jnp.float32)


def _gather_dot(cand, embed_table, emb):
    return pl.pallas_call(
        _gather_dot_body,
        grid_spec=pltpu.PrefetchScalarGridSpec(
            num_scalar_prefetch=1,
            grid=(_B // _RB,),
            in_specs=[
                pl.BlockSpec(memory_space=pl.ANY),
                pl.BlockSpec((_RB, _E), lambda g, c: (g, 0)),
            ],
            out_specs=pl.BlockSpec((_RB, _NCAND), lambda g, c: (g, 0)),
            scratch_shapes=[
                pltpu.VMEM((_RB * _NCAND, _E), jnp.float32),
                pltpu.SemaphoreType.DMA,
            ],
        ),
        out_shape=jax.ShapeDtypeStruct((_B, _NCAND), jnp.float32),
    )(cand, embed_table, emb)


def _sc_body(logits_hbm, gy_hbm, cand_hbm, row_v, gy_v, cand_v):
    wid = lax.axis_index("s") * _NC + lax.axis_index("c")
    base = wid * _ROWS_PER_W
    lane = lax.iota(jnp.int32, _L)
    k_mask = lane < _K
    neg_inf = jnp.full((_L,), -jnp.inf, jnp.float32)

    pltpu.sync_copy(gy_hbm, gy_v)

    for r in range(_ROWS_PER_W):
        b = base + r
        pltpu.sync_copy(logits_hbm.at[pl.ds(b, 1)], row_v)

        def grp_body(g, carry):
            cval, cidx, thr = carry
            gbase = g * (_L * _CHUNKS_PER_GRP)
            m = row_v[0, pl.ds(gbase, _L)]
            for j in range(1, _CHUNKS_PER_GRP):
                m = jnp.maximum(m, row_v[0, pl.ds(gbase + j * _L, _L)])
            gmax = jnp.max(m)

            def merge(c3):
                cv, ci, _ = c3
                for j in range(_CHUNKS_PER_GRP):
                    v = row_v[0, pl.ds(gbase + j * _L, _L)]
                    vi = gbase + j * _L + lane
                    sv, si = plsc.sort_key_val(v, vi, descending=True)
                    rv = lax.rev(sv, (0,))
                    ri = lax.rev(si, (0,))
                    take = rv > cv
                    nv = jnp.where(take, rv, cv)
                    ni = jnp.where(take, ri, ci)
                    cv, ci = plsc.sort_key_val(nv, ni, descending=True)
                new_thr = jnp.min(jnp.where(k_mask, cv, jnp.inf))
                return cv, ci, new_thr

            return lax.cond(gmax > thr, merge, lambda c3: c3,
                            (cval, cidx, thr))

        _, cidx, _ = lax.fori_loop(
            0, _GRPS, grp_body,
            (neg_inf, jnp.zeros((_L,), jnp.int32), -jnp.inf))

        safe_idx = jnp.where(k_mask, cidx, 0)
        ev = plsc.load_gather(gy_v, [safe_idx * 2])
        ov = plsc.load_gather(gy_v, [safe_idx * 2 + 1])
        plsc.store_scatter(cand_v, [r * _NCAND + lane * 2], ev, mask=k_mask)
        plsc.store_scatter(cand_v, [r * _NCAND + lane * 2 + 1], ov,
                           mask=k_mask)

    pltpu.sync_copy(
        cand_v, cand_hbm.at[pl.ds(base * _NCAND, _ROWS_PER_W * _NCAND)])


_sc_topk_route = functools.partial(
    pl.kernel,
    mesh=plsc.VectorSubcoreMesh(core_axis_name="c", subcore_axis_name="s"),
    out_type=jax.ShapeDtypeStruct((_B * _NCAND,), jnp.int32),
    scratch_types=[
        pltpu.VMEM((1, _C), jnp.float32),
        pltpu.VMEM((_NL,), jnp.int32),
        pltpu.VMEM((_ROWS_PER_W * _NCAND,), jnp.int32),
    ],
    compiler_params=pltpu.CompilerParams(
        needs_layout_passes=False, use_tc_tiling_on_sc=False),
)(_sc_body)


@jax.jit
def kernel(hidden_states, labels, W1, b1, W2, b2, group_y, embed_table):
    del labels
    cls_feats = jnp.concatenate(
        [hidden_states[-i][:, 0] for i in range(1, _FEATURE_LAYERS + 1)],
        axis=-1)
    logits = _matmul_bias(cls_feats, W1, b1, 1024)
    cand = _sc_topk_route(logits, group_y.reshape(-1))
    embT = _matmulT_bias(W2.T, logits, b2, 512)
    return _gather_dot(cand, embed_table, embT.T)
